# baseline (speedup 1.0000x reference)
import functools

import jax
import jax.numpy as jnp
from jax import lax
from jax.experimental import pallas as pl
from jax.experimental.pallas import tpu as pltpu
from jax.experimental.pallas import tpu_sc as plsc

_BN = 256
_KC = 2048


def _vq_argmin_body(x_ref, w_ref, sx_ref, sw_ref, idx_ref, loss_ref):
    x = x_ref[...]
    sx = sx_ref[...]
    bn = x.shape[0]
    num_k = w_ref.shape[0]

    def chunk(k, carry):
        best_d, best_i = carry
        w_chunk = w_ref[pl.ds(k * _KC, _KC), :]
        sw_chunk = sw_ref[pl.ds(k * _KC, _KC)]
        m = lax.dot_general(
            x, w_chunk, (((1,), (1,)), ((), ())),
            preferred_element_type=jnp.float32,
        )
        d = (sx[:, None] + sw_chunk[None, :]) - 2.0 * m
        local_d = jnp.min(d, axis=1)
        col = lax.broadcasted_iota(jnp.int32, d.shape, 1)
        local_i = jnp.min(
            jnp.where(d == local_d[:, None], col, jnp.int32(2**30)), axis=1
        )
        take = local_d < best_d
        best_d = jnp.where(take, local_d, best_d)
        best_i = jnp.where(take, local_i + k * _KC, best_i)
        return best_d, best_i

    init = (jnp.full((bn,), jnp.inf, jnp.float32),
            jnp.zeros((bn,), jnp.int32))
    best_d, best_i = lax.fori_loop(0, num_k // _KC, chunk, init)
    idx_ref[...] = best_i

    @pl.when(pl.program_id(0) == 0)
    def _():
        loss_ref[0, 0] = 0.0

    loss_ref[0, 0] += jnp.sum(best_d)


def _argmin_call(inputs, weight, sx, sw):
    n, d = inputs.shape
    k = weight.shape[0]
    return pl.pallas_call(
        _vq_argmin_body,
        grid=(n // _BN,),
        in_specs=[
            pl.BlockSpec((_BN, d), lambda i: (i, 0)),
            pl.BlockSpec((k, d), lambda i: (0, 0)),
            pl.BlockSpec((_BN,), lambda i: (i,)),
            pl.BlockSpec((k,), lambda i: (0,)),
        ],
        out_specs=[
            pl.BlockSpec((_BN,), lambda i: (i,)),
            pl.BlockSpec(memory_space=pltpu.SMEM),
        ],
        out_shape=[
            jax.ShapeDtypeStruct((n,), jnp.int32),
            jax.ShapeDtypeStruct((1, 1), jnp.float32),
        ],
    )(inputs, weight, sx, sw)


def _gather_call(weight, idx):
    n = idx.shape[0]
    k, d = weight.shape
    info = plsc.get_sparse_core_info()
    nc, ns = info.num_cores, info.num_subcores
    nw = nc * ns
    b_per_w = n // nw
    ch = 256
    n_chunks = b_per_w // ch
    mesh = plsc.VectorSubcoreMesh(core_axis_name="c", subcore_axis_name="s")

    @functools.partial(
        pl.kernel, mesh=mesh,
        out_type=jax.ShapeDtypeStruct((n, d), jnp.float32),
        scratch_types=[
            pltpu.VMEM((ch,), jnp.int32),
            pltpu.VMEM((ch, d), jnp.float32),
            pltpu.SemaphoreType.DMA,
        ],
    )
    def gather_k(table_hbm, idx_hbm, out_hbm, idx_v, rows_v, sem):
        wid = lax.axis_index("s") * nc + lax.axis_index("c")
        for c in range(n_chunks):
            base = wid * b_per_w + c * ch
            pltpu.sync_copy(idx_hbm.at[pl.ds(base, ch)], idx_v)
            pltpu.async_copy(table_hbm.at[idx_v], rows_v, sem).wait()
            pltpu.sync_copy(rows_v, out_hbm.at[pl.ds(base, ch)])

    return gather_k(weight, idx)


def kernel(inputs, weight):
    n, d = inputs.shape
    sx = jnp.sum(inputs * inputs, axis=1)
    sw = jnp.sum(weight * weight, axis=1)
    idx, loss_sum = _argmin_call(inputs, weight, sx, sw)
    quantized = _gather_call(weight, idx)
    m = loss_sum[0, 0] / jnp.float32(n * d)
    loss = m + jnp.float32(0.25) * m
    encoding_indices = idx.reshape(n, 1, 1)
    return (quantized, loss, encoding_indices)
